# Initial kernel scaffold; baseline (speedup 1.0000x reference)
import functools
import math

import jax
import jax.numpy as jnp
from jax import lax
from jax.experimental import pallas as pl
from jax.experimental.pallas import tpu as pltpu
from jax.experimental.pallas import tpu_sc as plsc

BATCH = 16384
EMB = 32
HID = 64
EPS = 1e-5
_BN_SCALE = 1.0 / math.sqrt(1.0 + EPS)

_NC = 2
_NS = 16
_NW = _NC * _NS
_BPW = BATCH // _NW
_CHUNK = 128
_NCHUNK = _BPW // _CHUNK


def _sc_gather(users3, movies3, user_table, movie_table):
    mesh = plsc.VectorSubcoreMesh(core_axis_name="c", subcore_axis_name="s")

    @functools.partial(
        pl.kernel,
        mesh=mesh,
        out_type=[
            jax.ShapeDtypeStruct((BATCH, EMB), jnp.float32),
            jax.ShapeDtypeStruct((BATCH, EMB), jnp.float32),
        ],
        scratch_types=[
            pltpu.VMEM((_NCHUNK, _CHUNK), jnp.int32),
            pltpu.VMEM((_NCHUNK, _CHUNK), jnp.int32),
            pltpu.VMEM((_BPW, EMB), jnp.float32),
            pltpu.VMEM((_BPW, EMB), jnp.float32),
            pltpu.SemaphoreType.DMA,
        ],
    )
    def gather_kernel(u_hbm, m_hbm, ut_hbm, mt_hbm, ue_out, me_out,
                      uidx, midx, urows, mrows, sem):
        wid = lax.axis_index("s") * _NC + lax.axis_index("c")
        base = wid * _BPW
        pltpu.sync_copy(u_hbm.at[wid], uidx)
        pltpu.sync_copy(m_hbm.at[wid], midx)
        copies = []
        for j in range(_NCHUNK):
            dst = pl.ds(j * _CHUNK, _CHUNK)
            copies.append(pltpu.async_copy(ut_hbm.at[uidx.at[j]], urows.at[dst], sem))
            copies.append(pltpu.async_copy(mt_hbm.at[midx.at[j]], mrows.at[dst], sem))
        for c in copies:
            c.wait()
        pltpu.sync_copy(urows, ue_out.at[pl.ds(base, _BPW)])
        pltpu.sync_copy(mrows, me_out.at[pl.ds(base, _BPW)])

    return gather_kernel(users3, movies3, user_table, movie_table)


def _mlp_body(ue_ref, me_ref, w1_ref, b1_ref, g1_ref, beta1_ref,
              w2_ref, b2_ref, g2_ref, beta2_ref, wo_ref, bo_ref, out_ref):
    f32 = jnp.float32
    w1 = w1_ref[...]
    h = lax.dot_general(ue_ref[...], w1[:, :EMB],
                        (((1,), (1,)), ((), ())), preferred_element_type=f32)
    h = h + lax.dot_general(me_ref[...], w1[:, EMB:],
                            (((1,), (1,)), ((), ())), preferred_element_type=f32)
    h = jnp.maximum(h + b1_ref[...], 0.0)
    s1 = g1_ref[...] * _BN_SCALE
    w2s = w2_ref[...] * s1
    h2 = lax.dot_general(h, w2s, (((1,), (1,)), ((), ())),
                         preferred_element_type=f32)
    h2 = h2 + b2_ref[...] + lax.dot_general(
        beta1_ref[...], w2_ref[...], (((1,), (1,)), ((), ())),
        preferred_element_type=f32)
    h2 = jnp.maximum(h2, 0.0)
    s2 = g2_ref[...] * _BN_SCALE
    wos = wo_ref[...] * s2
    out = jnp.sum(h2 * wos, axis=1, keepdims=True)
    out_ref[...] = out + (jnp.sum(beta2_ref[...] * wos, axis=1, keepdims=True)
                          + bo_ref[...])


def _tc_mlp(ue, me, W1, b1, g1, beta1, W2, b2, g2, beta2, Wo, bo):
    BM = 2048
    grid = (BATCH // BM,)
    row = lambda i: (i, 0)
    fixed = lambda i: (0, 0)
    return pl.pallas_call(
        _mlp_body,
        grid=grid,
        in_specs=[
            pl.BlockSpec((BM, EMB), row),
            pl.BlockSpec((BM, EMB), row),
            pl.BlockSpec((HID, 2 * EMB), fixed),
            pl.BlockSpec((1, HID), fixed),
            pl.BlockSpec((1, HID), fixed),
            pl.BlockSpec((1, HID), fixed),
            pl.BlockSpec((HID, HID), fixed),
            pl.BlockSpec((1, HID), fixed),
            pl.BlockSpec((1, HID), fixed),
            pl.BlockSpec((1, HID), fixed),
            pl.BlockSpec((1, HID), fixed),
            pl.BlockSpec((1, 1), fixed),
        ],
        out_specs=pl.BlockSpec((BM, 1), row),
        out_shape=jax.ShapeDtypeStruct((BATCH, 1), jnp.float32),
    )(ue, me, W1, b1, g1, beta1, W2, b2, g2, beta2, Wo, bo)


def kernel(users, movies, user_table, movie_table,
           W1, b1, g1, beta1, W2, b2, g2, beta2, Wo, bo):
    users3 = users.astype(jnp.int32).reshape(_NW, _NCHUNK, _CHUNK)
    movies3 = movies.astype(jnp.int32).reshape(_NW, _NCHUNK, _CHUNK)
    ue, me = _sc_gather(users3, movies3, user_table, movie_table)
    return _tc_mlp(
        ue, me, W1,
        b1.reshape(1, HID), g1.reshape(1, HID), beta1.reshape(1, HID),
        W2, b2.reshape(1, HID), g2.reshape(1, HID), beta2.reshape(1, HID),
        Wo.reshape(1, HID), bo.reshape(1, 1),
    )


# speedup vs baseline: 1.6144x; 1.6144x over previous
import functools
import math

import jax
import jax.numpy as jnp
from jax import lax
from jax.experimental import pallas as pl
from jax.experimental.pallas import tpu as pltpu
from jax.experimental.pallas import tpu_sc as plsc

BATCH = 16384
EMB = 32
HID = 64
EPS = 1e-5
_BN_SCALE = 1.0 / math.sqrt(1.0 + EPS)

_NC = 2
_NS = 16
_NW = _NC * _NS
_BPW = BATCH // _NW
_CH = 256
_TBL = 8192


def _tp_body(ut_ref, mt_ref, uo_ref, mo_ref):
    uo_ref[...] = ut_ref[...].T
    mo_ref[...] = mt_ref[...].T


def _tc_transpose(ut_t, mt_t):
    n = ut_t.shape[1]
    grid = (pl.cdiv(n, _TBL),)
    out = jax.ShapeDtypeStruct((n, EMB), jnp.float32)
    return pl.pallas_call(
        _tp_body,
        grid=grid,
        in_specs=[pl.BlockSpec((EMB, _TBL), lambda i: (0, i)),
                  pl.BlockSpec((EMB, _TBL), lambda i: (0, i))],
        out_specs=[pl.BlockSpec((_TBL, EMB), lambda i: (i, 0)),
                   pl.BlockSpec((_TBL, EMB), lambda i: (i, 0))],
        out_shape=[out, out],
    )(ut_t, mt_t)


def _sc_gather(users, movies, user_table, movie_table):
    mesh = plsc.VectorSubcoreMesh(core_axis_name="c", subcore_axis_name="s")

    @functools.partial(
        pl.kernel,
        mesh=mesh,
        out_type=[
            jax.ShapeDtypeStruct((BATCH, EMB), jnp.float32),
            jax.ShapeDtypeStruct((BATCH, EMB), jnp.float32),
        ],
        scratch_types=[
            pltpu.VMEM((_CH,), jnp.int32),
            pltpu.VMEM((_CH,), jnp.int32),
            pltpu.VMEM((_CH, EMB), jnp.float32),
            pltpu.VMEM((_CH, EMB), jnp.float32),
            pltpu.SemaphoreType.DMA,
        ],
    )
    def gather_kernel(u_hbm, m_hbm, ut_hbm, mt_hbm, ue_out, me_out,
                      uidx, midx, urows, mrows, sem):
        wid = lax.axis_index("s") * _NC + lax.axis_index("c")
        for c in range(_BPW // _CH):
            base = wid * _BPW + c * _CH
            pltpu.sync_copy(u_hbm.at[pl.ds(base, _CH)], uidx)
            pltpu.sync_copy(m_hbm.at[pl.ds(base, _CH)], midx)

            def body(g, carry):
                gbase = g * 16
                vu = uidx[pl.ds(gbase, 16)]
                vm = midx[pl.ds(gbase, 16)]
                for j in range(16):
                    pltpu.async_copy(ut_hbm.at[vu[j]], urows.at[gbase + j], sem)
                    pltpu.async_copy(mt_hbm.at[vm[j]], mrows.at[gbase + j], sem)
                return carry

            lax.fori_loop(0, _CH // 16, body, 0)
            pltpu.make_async_copy(ut_hbm.at[pl.ds(0, _CH)], urows, sem).wait()
            pltpu.make_async_copy(mt_hbm.at[pl.ds(0, _CH)], mrows, sem).wait()
            pltpu.sync_copy(urows, ue_out.at[pl.ds(base, _CH)])
            pltpu.sync_copy(mrows, me_out.at[pl.ds(base, _CH)])

    return gather_kernel(users, movies, user_table, movie_table)


def _mlp_body(ue_ref, me_ref, w1_ref, b1_ref, g1_ref, beta1_ref,
              w2_ref, b2_ref, g2_ref, beta2_ref, wo_ref, bo_ref, out_ref):
    f32 = jnp.float32
    w1 = w1_ref[...]
    h = lax.dot_general(ue_ref[...], w1[:, :EMB],
                        (((1,), (1,)), ((), ())), preferred_element_type=f32)
    h = h + lax.dot_general(me_ref[...], w1[:, EMB:],
                            (((1,), (1,)), ((), ())), preferred_element_type=f32)
    h = jnp.maximum(h + b1_ref[...], 0.0)
    s1 = g1_ref[...] * _BN_SCALE
    w2s = w2_ref[...] * s1
    h2 = lax.dot_general(h, w2s, (((1,), (1,)), ((), ())),
                         preferred_element_type=f32)
    h2 = h2 + b2_ref[...] + lax.dot_general(
        beta1_ref[...], w2_ref[...], (((1,), (1,)), ((), ())),
        preferred_element_type=f32)
    h2 = jnp.maximum(h2, 0.0)
    s2 = g2_ref[...] * _BN_SCALE
    wos = wo_ref[...] * s2
    out = jnp.sum(h2 * wos, axis=1, keepdims=True)
    out_ref[...] = out + (jnp.sum(beta2_ref[...] * wos, axis=1, keepdims=True)
                          + bo_ref[...])


def _tc_mlp(ue, me, W1, b1, g1, beta1, W2, b2, g2, beta2, Wo, bo):
    BM = 2048
    grid = (BATCH // BM,)
    row = lambda i: (i, 0)
    fixed = lambda i: (0, 0)
    return pl.pallas_call(
        _mlp_body,
        grid=grid,
        in_specs=[
            pl.BlockSpec((BM, EMB), row),
            pl.BlockSpec((BM, EMB), row),
            pl.BlockSpec((HID, 2 * EMB), fixed),
            pl.BlockSpec((1, HID), fixed),
            pl.BlockSpec((1, HID), fixed),
            pl.BlockSpec((1, HID), fixed),
            pl.BlockSpec((HID, HID), fixed),
            pl.BlockSpec((1, HID), fixed),
            pl.BlockSpec((1, HID), fixed),
            pl.BlockSpec((1, HID), fixed),
            pl.BlockSpec((1, HID), fixed),
            pl.BlockSpec((1, 1), fixed),
        ],
        out_specs=pl.BlockSpec((BM, 1), row),
        out_shape=jax.ShapeDtypeStruct((BATCH, 1), jnp.float32),
    )(ue, me, W1, b1, g1, beta1, W2, b2, g2, beta2, Wo, bo)


def kernel(users, movies, user_table, movie_table,
           W1, b1, g1, beta1, W2, b2, g2, beta2, Wo, bo):
    ut_rm, mt_rm = _tc_transpose(user_table.T, movie_table.T)
    ue, me = _sc_gather(users.astype(jnp.int32), movies.astype(jnp.int32),
                        ut_rm, mt_rm)
    return _tc_mlp(
        ue, me, W1,
        b1.reshape(1, HID), g1.reshape(1, HID), beta1.reshape(1, HID),
        W2, b2.reshape(1, HID), g2.reshape(1, HID), beta2.reshape(1, HID),
        Wo.reshape(1, HID), bo.reshape(1, 1),
    )


# speedup vs baseline: 2.0376x; 1.2621x over previous
import functools
import math

import jax
import jax.numpy as jnp
from jax import lax
from jax.experimental import pallas as pl
from jax.experimental.pallas import tpu as pltpu
from jax.experimental.pallas import tpu_sc as plsc

BATCH = 16384
EMB = 32
HID = 64
EPS = 1e-5
_BN_SCALE = 1.0 / math.sqrt(1.0 + EPS)
_PK = 128 // EMB

_NC = 2
_NS = 16
_NW = _NC * _NS
_BPW = BATCH // _NW
_CH = 256
_TBL = 8192


def _tp_body(ut_ref, mt_ref, uo_ref, mo_ref):
    bk = _TBL // _PK
    lane = lax.broadcasted_iota(jnp.int32, (EMB, _PK * EMB), 1)
    feat = lax.broadcasted_iota(jnp.int32, (EMB, _PK * EMB), 0)
    for ref, o in ((ut_ref, uo_ref), (mt_ref, mo_ref)):
        acc = None
        for a in range(_PK):
            e = (lane == feat + EMB * a).astype(jnp.float32)
            t = lax.dot_general(ref[:, a * bk:(a + 1) * bk], e,
                                (((0,), (0,)), ((), ())),
                                preferred_element_type=jnp.float32)
            acc = t if acc is None else acc + t
        o[...] = acc


def _tc_transpose_pack(ut_t, mt_t):
    n = ut_t.shape[1]
    bk = _TBL // _PK
    grid = (pl.cdiv(n, _TBL),)
    out = jax.ShapeDtypeStruct((n // _PK, _PK * EMB), jnp.float32)
    return pl.pallas_call(
        _tp_body,
        grid=grid,
        in_specs=[pl.BlockSpec((EMB, _TBL), lambda i: (0, i)),
                  pl.BlockSpec((EMB, _TBL), lambda i: (0, i))],
        out_specs=[pl.BlockSpec((bk, _PK * EMB), lambda i: (i, 0)),
                   pl.BlockSpec((bk, _PK * EMB), lambda i: (i, 0))],
        out_shape=[out, out],
        compiler_params=pltpu.CompilerParams(fuse_transposed_lhs_in_matmul=True),
    )(ut_t, mt_t)


def _sc_gather(users, movies, ut_pk, mt_pk):
    mesh = plsc.VectorSubcoreMesh(core_axis_name="c", subcore_axis_name="s")

    @functools.partial(
        pl.kernel,
        mesh=mesh,
        out_type=[
            jax.ShapeDtypeStruct((BATCH, _PK * EMB), jnp.float32),
            jax.ShapeDtypeStruct((BATCH, _PK * EMB), jnp.float32),
        ],
        scratch_types=[
            pltpu.VMEM((_CH,), jnp.int32),
            pltpu.VMEM((_CH,), jnp.int32),
            pltpu.VMEM((_CH, _PK * EMB), jnp.float32),
            pltpu.VMEM((_CH, _PK * EMB), jnp.float32),
            pltpu.SemaphoreType.DMA,
        ],
    )
    def gather_kernel(u_hbm, m_hbm, ut_hbm, mt_hbm, ue_out, me_out,
                      uidx, midx, urows, mrows, sem):
        wid = lax.axis_index("s") * _NC + lax.axis_index("c")
        for c in range(_BPW // _CH):
            base = wid * _BPW + c * _CH
            pltpu.sync_copy(u_hbm.at[pl.ds(base, _CH)], uidx)
            pltpu.sync_copy(m_hbm.at[pl.ds(base, _CH)], midx)

            def body(g, carry):
                gbase = g * 16
                vu = uidx[pl.ds(gbase, 16)]
                vm = midx[pl.ds(gbase, 16)]
                for j in range(16):
                    pltpu.async_copy(ut_hbm.at[vu[j]], urows.at[gbase + j], sem)
                    pltpu.async_copy(mt_hbm.at[vm[j]], mrows.at[gbase + j], sem)
                return carry

            lax.fori_loop(0, _CH // 16, body, 0)
            pltpu.make_async_copy(ut_hbm.at[pl.ds(0, _CH)], urows, sem).wait()
            pltpu.make_async_copy(mt_hbm.at[pl.ds(0, _CH)], mrows, sem).wait()
            pltpu.sync_copy(urows, ue_out.at[pl.ds(base, _CH)])
            pltpu.sync_copy(mrows, me_out.at[pl.ds(base, _CH)])

    return gather_kernel(users, movies, ut_pk, mt_pk)


def _mlp_body(up_ref, mp_ref, au_ref, am_ref, w1_ref, b1_ref, g1_ref, beta1_ref,
              w2_ref, b2_ref, g2_ref, beta2_ref, wo_ref, bo_ref, out_ref):
    f32 = jnp.float32
    lane_grp = lax.broadcasted_iota(jnp.int32, (1, _PK * EMB), 1) // EMB
    up = jnp.where(au_ref[...] == lane_grp, up_ref[...], 0.0)
    mp = jnp.where(am_ref[...] == lane_grp, mp_ref[...], 0.0)
    w1 = w1_ref[...]
    w1u = jnp.concatenate([w1[:, :EMB]] * _PK, axis=1)
    w1m = jnp.concatenate([w1[:, EMB:]] * _PK, axis=1)
    h = lax.dot_general(up, w1u, (((1,), (1,)), ((), ())),
                        preferred_element_type=f32)
    h = h + lax.dot_general(mp, w1m, (((1,), (1,)), ((), ())),
                            preferred_element_type=f32)
    h = jnp.maximum(h + b1_ref[...], 0.0)
    s1 = g1_ref[...] * _BN_SCALE
    w2s = w2_ref[...] * s1
    h2 = lax.dot_general(h, w2s, (((1,), (1,)), ((), ())),
                         preferred_element_type=f32)
    h2 = h2 + b2_ref[...] + lax.dot_general(
        beta1_ref[...], w2_ref[...], (((1,), (1,)), ((), ())),
        preferred_element_type=f32)
    h2 = jnp.maximum(h2, 0.0)
    s2 = g2_ref[...] * _BN_SCALE
    wos = wo_ref[...] * s2
    out = jnp.sum(h2 * wos, axis=1, keepdims=True)
    out_ref[...] = out + (jnp.sum(beta2_ref[...] * wos, axis=1, keepdims=True)
                          + bo_ref[...])


def _tc_mlp(up, mp, au, am, W1, b1, g1, beta1, W2, b2, g2, beta2, Wo, bo):
    BM = 2048
    grid = (BATCH // BM,)
    row = lambda i: (i, 0)
    fixed = lambda i: (0, 0)
    return pl.pallas_call(
        _mlp_body,
        grid=grid,
        in_specs=[
            pl.BlockSpec((BM, _PK * EMB), row),
            pl.BlockSpec((BM, _PK * EMB), row),
            pl.BlockSpec((BM, 1), row),
            pl.BlockSpec((BM, 1), row),
            pl.BlockSpec((HID, 2 * EMB), fixed),
            pl.BlockSpec((1, HID), fixed),
            pl.BlockSpec((1, HID), fixed),
            pl.BlockSpec((1, HID), fixed),
            pl.BlockSpec((HID, HID), fixed),
            pl.BlockSpec((1, HID), fixed),
            pl.BlockSpec((1, HID), fixed),
            pl.BlockSpec((1, HID), fixed),
            pl.BlockSpec((1, HID), fixed),
            pl.BlockSpec((1, 1), fixed),
        ],
        out_specs=pl.BlockSpec((BM, 1), row),
        out_shape=jax.ShapeDtypeStruct((BATCH, 1), jnp.float32),
    )(up, mp, au, am, W1, b1, g1, beta1, W2, b2, g2, beta2, Wo, bo)


def kernel(users, movies, user_table, movie_table,
           W1, b1, g1, beta1, W2, b2, g2, beta2, Wo, bo):
    users = users.astype(jnp.int32)
    movies = movies.astype(jnp.int32)
    ut_pk, mt_pk = _tc_transpose_pack(user_table.T, movie_table.T)
    bk = _TBL // _PK
    urow = (users // _TBL) * bk + (users % bk)
    mrow = (movies // _TBL) * bk + (movies % bk)
    ugrp = (users % _TBL) // bk
    mgrp = (movies % _TBL) // bk
    up, mp = _sc_gather(urow, mrow, ut_pk, mt_pk)
    return _tc_mlp(
        up, mp, ugrp.reshape(BATCH, 1), mgrp.reshape(BATCH, 1),
        W1,
        b1.reshape(1, HID), g1.reshape(1, HID), beta1.reshape(1, HID),
        W2, b2.reshape(1, HID), g2.reshape(1, HID), beta2.reshape(1, HID),
        Wo.reshape(1, HID), bo.reshape(1, 1),
    )
